# baseline (speedup 1.0000x reference)
import functools

import jax
import jax.numpy as jnp
import numpy as np
from jax import lax
from jax.experimental import pallas as pl
from jax.experimental.pallas import tpu as pltpu
from jax.experimental.pallas import tpu_sc as plsc

INPUT_DIM = 100000
OUTPUT_DIM = 64
BATCH = 4096
SEQ = 200
HALF = SEQ // 2
SUPER = 2 * SEQ
SCALE = float(np.sqrt(np.float32(OUTPUT_DIM)))


def _pos_encoding(position, d_model):
    i = np.arange(d_model)[np.newaxis, :]
    angle_rates = 1 / np.power(10000, 2 * (i // 2) / np.float32(d_model))
    angle_rads = np.arange(position)[:, np.newaxis] * angle_rates
    angle_rads[:, 0::2] = np.sin(angle_rads[:, 0::2])
    angle_rads[:, 1::2] = np.cos(angle_rads[:, 1::2])
    return np.asarray(angle_rads, dtype=np.float32)


_POS = _pos_encoding(SEQ, OUTPUT_DIM)


@functools.cache
def _build_kernel(nc, ns):
    nw = nc * ns
    total_rows = BATCH * SEQ
    rows_per_w = total_rows // nw
    supers_per_w = rows_per_w // SUPER
    chunks_per_w = rows_per_w // HALF

    mesh = plsc.VectorSubcoreMesh(
        core_axis_name="c", subcore_axis_name="s",
        num_cores=nc, num_subcores=ns)

    @functools.partial(
        pl.kernel,
        out_type=jax.ShapeDtypeStruct((total_rows // 2, 2 * OUTPUT_DIM),
                                      jnp.float32),
        mesh=mesh,
        scratch_types=[
            pltpu.VMEM((HALF, 2 * OUTPUT_DIM), jnp.float32),
            [pltpu.VMEM((4, HALF), jnp.int32) for _ in range(2)],
            [pltpu.VMEM((SUPER, OUTPUT_DIM), jnp.float32) for _ in range(2)],
            [pltpu.VMEM((SEQ, 2 * OUTPUT_DIM), jnp.float32) for _ in range(2)],
            [pltpu.SemaphoreType.DMA for _ in range(2)],
            [pltpu.SemaphoreType.DMA for _ in range(2)],
        ],
        compiler_params=pltpu.CompilerParams(use_tc_tiling_on_sc=False),
    )
    def emb_kernel(idx_hbm, table_hbm, pos_hbm, out_hbm, pos_v,
                   idxbufs, gbufs, sbufs, gsems, ssems):
        wid = lax.axis_index("s") * nc + lax.axis_index("c")
        pltpu.sync_copy(pos_hbm, pos_v)
        base = wid * supers_per_w

        def stage_idx(t, ib):
            pltpu.sync_copy(idx_hbm.at[wid, pl.ds(4 * t, 4)], ib)

        def gather_copies(ib, gb, gsem):
            return [pltpu.make_async_copy(
                table_hbm.at[ib.at[k]], gb.at[pl.ds(k * HALF, HALF)], gsem)
                for k in range(4)]

        def store_copy(t, sb, ssem):
            return pltpu.make_async_copy(
                sb, out_hbm.at[pl.ds((base + t) * SEQ, SEQ)], ssem)

        for b in range(2):
            stage_idx(b, idxbufs[b])
            for cp in gather_copies(idxbufs[b], gbufs[b], gsems[b]):
                cp.start()

        def body(m, carry):
            t0 = 2 * m
            for b in range(2):
                t = t0 + b
                ib, gb, sb = idxbufs[b], gbufs[b], sbufs[b]
                gsem, ssem = gsems[b], ssems[b]
                for cp in gather_copies(ib, gb, gsem):
                    cp.wait()

                @pl.when(t >= 2)
                def _():
                    store_copy(t, sb, ssem).wait()

                for half in range(2):
                    go, so = half * SEQ, half * HALF

                    @functools.partial(plsc.parallel_loop, 0, HALF, unroll=2)
                    def _(j):
                        for e in range(2):
                            for c in range(OUTPUT_DIM // 16):
                                sl = pl.ds(c * 16, 16)
                                osl = pl.ds(e * OUTPUT_DIM + c * 16, 16)
                                sb[so + j, osl] = (
                                    gb[go + 2 * j + e, sl] * SCALE
                                    + pos_v[j, osl])

                store_copy(t, sb, ssem).start()

                @pl.when(t + 2 < supers_per_w)
                def _():
                    stage_idx(t + 2, ib)
                    for cp in gather_copies(ib, gb, gsem):
                        cp.start()
            return carry

        lax.fori_loop(0, supers_per_w // 2, body, 0)
        for b in range(2):
            store_copy(supers_per_w - 2 + b, sbufs[b], ssems[b]).wait()

    return emb_kernel


def kernel(x, table):
    info = plsc.get_sparse_core_info()
    nc, ns = info.num_cores, info.num_subcores
    nw = nc * ns
    idx = x.reshape(nw, (BATCH * SEQ) // nw // HALF, HALF)
    pos = jnp.asarray(_POS.reshape(HALF, 2 * OUTPUT_DIM))
    out2 = _build_kernel(nc, ns)(idx, table, pos)
    return out2.reshape(BATCH, SEQ, OUTPUT_DIM)


# speedup vs baseline: 1.0010x; 1.0010x over previous
import functools

import jax
import jax.numpy as jnp
import numpy as np
from jax import lax
from jax.experimental import pallas as pl
from jax.experimental.pallas import tpu as pltpu
from jax.experimental.pallas import tpu_sc as plsc

INPUT_DIM = 100000
OUTPUT_DIM = 64
BATCH = 4096
SEQ = 200
HALF = SEQ // 2
SCALE = float(np.sqrt(np.float32(OUTPUT_DIM)))


def _pos_encoding(position, d_model):
    i = np.arange(d_model)[np.newaxis, :]
    angle_rates = 1 / np.power(10000, 2 * (i // 2) / np.float32(d_model))
    angle_rads = np.arange(position)[:, np.newaxis] * angle_rates
    angle_rads[:, 0::2] = np.sin(angle_rads[:, 0::2])
    angle_rads[:, 1::2] = np.cos(angle_rads[:, 1::2])
    return np.asarray(angle_rads, dtype=np.float32)


_POS = _pos_encoding(SEQ, OUTPUT_DIM)


@functools.cache
def _build_kernel(nc, ns):
    nw = nc * ns
    total_rows = BATCH * SEQ
    rows_per_w = total_rows // nw
    seqs_per_w = rows_per_w // SEQ
    chunks_per_w = rows_per_w // HALF

    mesh = plsc.VectorSubcoreMesh(
        core_axis_name="c", subcore_axis_name="s",
        num_cores=nc, num_subcores=ns)

    @functools.partial(
        pl.kernel,
        out_type=jax.ShapeDtypeStruct((BATCH, SEQ, OUTPUT_DIM), jnp.float32),
        mesh=mesh,
        scratch_types=[
            pltpu.VMEM((chunks_per_w, HALF), jnp.int32),
            pltpu.VMEM((SEQ, OUTPUT_DIM), jnp.float32),
            pltpu.VMEM((SEQ, OUTPUT_DIM), jnp.float32),
            pltpu.VMEM((SEQ, OUTPUT_DIM), jnp.float32),
            pltpu.VMEM((SEQ, OUTPUT_DIM), jnp.float32),
            pltpu.VMEM((SEQ, OUTPUT_DIM), jnp.float32),
            pltpu.SemaphoreType.DMA,
            pltpu.SemaphoreType.DMA,
            pltpu.SemaphoreType.DMA,
            pltpu.SemaphoreType.DMA,
        ],
        compiler_params=pltpu.CompilerParams(use_tc_tiling_on_sc=False),
    )
    def emb_kernel(idx_hbm, table_hbm, pos_hbm, out_hbm, idx_v, pos_v,
                   g0, g1, s0, s1, gsem0, gsem1, ssem0, ssem1):
        wid = lax.axis_index("s") * nc + lax.axis_index("c")
        pltpu.sync_copy(idx_hbm.at[wid], idx_v)
        pltpu.sync_copy(pos_hbm, pos_v)
        base = wid * seqs_per_w
        gbufs, sbufs = (g0, g1), (s0, s1)
        gsems, ssems = (gsem0, gsem1), (ssem0, ssem1)

        def fire_gather(si, gb, gsem):
            pltpu.async_copy(
                table_hbm.at[idx_v.at[2 * si]], gb.at[pl.ds(0, HALF)], gsem)
            pltpu.async_copy(
                table_hbm.at[idx_v.at[2 * si + 1]], gb.at[pl.ds(HALF, HALF)], gsem)

        def wait_gather(si, gb, gsem):
            pltpu.make_async_copy(
                table_hbm.at[idx_v.at[2 * si]], gb.at[pl.ds(0, HALF)], gsem).wait()
            pltpu.make_async_copy(
                table_hbm.at[idx_v.at[2 * si + 1]], gb.at[pl.ds(HALF, HALF)], gsem).wait()

        fire_gather(0, g0, gsem0)
        fire_gather(1, g1, gsem1)

        def body(j, carry):
            q = 2 * j
            for b in range(2):
                si = q + b
                gb, sb, gsem, ssem = gbufs[b], sbufs[b], gsems[b], ssems[b]
                wait_gather(si, gb, gsem)

                @pl.when(si >= 2)
                def _():
                    pltpu.make_async_copy(sb, out_hbm.at[base], ssem).wait()

                @functools.partial(plsc.parallel_loop, 0, SEQ, unroll=4)
                def _(r):
                    for c in range(OUTPUT_DIM // 16):
                        sl = pl.ds(c * 16, 16)
                        sb[r, sl] = gb[r, sl] * SCALE + pos_v[r, sl]

                pltpu.async_copy(sb, out_hbm.at[base + si], ssem)

                @pl.when(si + 2 < seqs_per_w)
                def _():
                    fire_gather(si + 2, gb, gsem)
            return carry

        lax.fori_loop(0, seqs_per_w // 2, body, 0)
        pltpu.make_async_copy(s0, out_hbm.at[base], ssem0).wait()
        pltpu.make_async_copy(s1, out_hbm.at[base], ssem1).wait()

    return emb_kernel


def kernel(x, table):
    info = plsc.get_sparse_core_info()
    nc, ns = info.num_cores, info.num_subcores
    nw = nc * ns
    idx = x.reshape(nw, (BATCH * SEQ) // nw // HALF, HALF)
    pos = jnp.asarray(_POS)
    return _build_kernel(nc, ns)(idx, table, pos)
